# baseline (speedup 1.0000x reference)
import functools

import jax
import jax.numpy as jnp
from jax import lax
from jax.experimental import pallas as pl
from jax.experimental.pallas import tpu as pltpu
from jax.experimental.pallas import tpu_sc as plsc

N = 10000
E = 320000
D = 128
L = 16
GROUP = 128
NC = 2
NS = 16
NW = NC * NS
NGT = 80
NH = 2
GH = NGT // NH
G_PAD = NW * NGT
E_PAD = G_PAD * GROUP
PCHUNK = 80
NPC = N // PCHUNK
PC_TILE = NPC // NS
PC_REM = NPC - PC_TILE * NS
MM_BLK = 2000


def _sc_layer(x, src, dst, wgt):
    mesh = plsc.VectorSubcoreMesh(core_axis_name="c", subcore_axis_name="s")

    @functools.partial(
        pl.kernel,
        out_type=jax.ShapeDtypeStruct((NC, N, D), jnp.float32),
        mesh=mesh,
        scratch_types=[
            pltpu.VMEM_SHARED((N, D), jnp.float32),
            pltpu.VMEM((GH, GROUP), jnp.int32),
            pltpu.VMEM((GH, GROUP), jnp.int32),
            pltpu.VMEM((GH, GROUP), jnp.float32),
            pltpu.VMEM((GROUP, D), jnp.float32),
            pltpu.VMEM((PCHUNK, D), jnp.float32),
            pltpu.SemaphoreType.DMA,
        ],
    )
    def sc_kernel(x_hbm, src_hbm, dst_hbm, w_hbm, out_hbm,
                  acc, sb, db, wb, rows, stage, sem):
        c = lax.axis_index("c")
        s = lax.axis_index("s")
        wid = c * NS + s
        g0 = wid * NGT

        def _zero(r, carry):
            for j in range(D // L):
                stage[r, pl.ds(j * L, L)] = jnp.zeros((L,), jnp.float32)
            return carry
        lax.fori_loop(0, PCHUNK, _zero, 0)
        pc0 = s * PC_TILE + jnp.minimum(s, PC_REM)
        pcnt = PC_TILE + jnp.where(s < PC_REM, 1, 0)

        def _zacc(k, carry):
            pltpu.sync_copy(stage, acc.at[pl.ds((pc0 + k) * PCHUNK, PCHUNK)])
            return carry
        lax.fori_loop(0, pcnt, _zacc, 0)
        plsc.subcore_barrier()

        def _half(h, carry):
            base = g0 + h * GH
            pltpu.sync_copy(src_hbm.at[pl.ds(base, GH)], sb)
            pltpu.sync_copy(dst_hbm.at[pl.ds(base, GH)], db)
            pltpu.sync_copy(w_hbm.at[pl.ds(base, GH)], wb)

            def _group(gg, carry2):
                pltpu.async_copy(x_hbm.at[sb.at[gg]], rows, sem).wait()

                def scale16(eb, carry3):
                    wv16 = wb[gg, pl.ds(eb * L, L)]
                    for lane in range(L):
                        wvb = jnp.full((L,), wv16[lane], jnp.float32)
                        e = eb * L + lane
                        for j in range(D // L):
                            rows[e, pl.ds(j * L, L)] = (
                                rows[e, pl.ds(j * L, L)] * wvb)
                    return carry3
                lax.fori_loop(0, GROUP // L, scale16, 0)

                pltpu.sync_copy(rows, acc.at[db.at[gg]], add=True)
                return carry2
            lax.fori_loop(0, GH, _group, 0)
            return carry
        lax.fori_loop(0, NH, _half, 0)

        plsc.subcore_barrier()

        def _pub(k, carry):
            r0 = (pc0 + k) * PCHUNK
            pltpu.sync_copy(acc.at[pl.ds(r0, PCHUNK)], stage)
            pltpu.sync_copy(stage, out_hbm.at[c, pl.ds(r0, PCHUNK)])
            return carry
        lax.fori_loop(0, pcnt, _pub, 0)

    return sc_kernel(x, src, dst, wgt)


def _mm_fused(p, b, W):
    def body(p_ref, b_ref, w_ref, o_ref):
        hs = p_ref[0] + p_ref[1]
        o_ref[...] = jnp.dot(hs, w_ref[...],
                             preferred_element_type=jnp.float32) + b_ref[...]
    return pl.pallas_call(
        body,
        grid=(N // MM_BLK,),
        in_specs=[pl.BlockSpec((NC, MM_BLK, D), lambda i: (0, i, 0)),
                  pl.BlockSpec((1, D), lambda i: (0, 0)),
                  pl.BlockSpec((D, D), lambda i: (0, 0))],
        out_specs=pl.BlockSpec((MM_BLK, D), lambda i: (i, 0)),
        out_shape=jax.ShapeDtypeStruct((N, D), jnp.float32),
    )(p, b, W)


def kernel(x, edge_index, edge_weight, W1, b1, W2, b2):
    pad = E_PAD - E
    src = jnp.concatenate(
        [edge_index[0], jnp.zeros((pad,), jnp.int32)]).reshape(G_PAD, GROUP)
    dst = jnp.concatenate(
        [edge_index[1], jnp.zeros((pad,), jnp.int32)]).reshape(G_PAD, GROUP)
    wgt = jnp.concatenate(
        [edge_weight, jnp.zeros((pad,), jnp.float32)]).reshape(G_PAD, GROUP)
    b1r = b1.reshape(1, D)
    b2r = b2.reshape(1, D)

    p1 = _sc_layer(x, src, dst, wgt)
    h1 = _mm_fused(p1, b1r, W1)
    p2 = _sc_layer(h1, src, dst, wgt)
    return _mm_fused(p2, b2r, W2)


# speedup vs baseline: 2.2323x; 2.2323x over previous
import functools

import jax
import jax.numpy as jnp
from jax import lax
from jax.experimental import pallas as pl
from jax.experimental.pallas import tpu as pltpu
from jax.experimental.pallas import tpu_sc as plsc

N = 10000
E = 320000
D = 128
L = 16
GROUP = 128
NC = 2
NS = 16
NW = NC * NS
NGT = 80
NH = 2
GH = NGT // NH
G_PAD = NW * NGT
E_PAD = G_PAD * GROUP
PCHUNK = 80
NPC = N // PCHUNK
PC_TILE = NPC // NS
PC_REM = NPC - PC_TILE * NS
MM_BLK = 2000


def _sc_layer(x, src, dst, wgt):
    mesh = plsc.VectorSubcoreMesh(core_axis_name="c", subcore_axis_name="s")

    @functools.partial(
        pl.kernel,
        out_type=jax.ShapeDtypeStruct((NC, N, D), jnp.float32),
        mesh=mesh,
        scratch_types=[
            pltpu.VMEM_SHARED((N, D), jnp.float32),
            pltpu.VMEM((GH, GROUP), jnp.int32),
            pltpu.VMEM((GH, GROUP), jnp.int32),
            pltpu.VMEM((GH, GROUP), jnp.float32),
            pltpu.VMEM((GROUP, D), jnp.float32),
            pltpu.VMEM((PCHUNK, D), jnp.float32),
            pltpu.SemaphoreType.DMA,
        ],
    )
    def sc_kernel(x_hbm, src_hbm, dst_hbm, w_hbm, out_hbm,
                  acc, sb, db, wb, rows, stage, sem):
        c = lax.axis_index("c")
        s = lax.axis_index("s")
        wid = c * NS + s
        g0 = wid * NGT

        def _zero(r, carry):
            for j in range(D // L):
                stage[r, pl.ds(j * L, L)] = jnp.zeros((L,), jnp.float32)
            return carry
        lax.fori_loop(0, PCHUNK, _zero, 0)
        pc0 = s * PC_TILE + jnp.minimum(s, PC_REM)
        pcnt = PC_TILE + jnp.where(s < PC_REM, 1, 0)

        def _zacc(k, carry):
            pltpu.sync_copy(stage, acc.at[pl.ds((pc0 + k) * PCHUNK, PCHUNK)])
            return carry
        lax.fori_loop(0, pcnt, _zacc, 0)
        plsc.subcore_barrier()

        def _half(h, carry):
            base = g0 + h * GH
            pltpu.sync_copy(src_hbm.at[pl.ds(base, GH)], sb)
            pltpu.sync_copy(dst_hbm.at[pl.ds(base, GH)], db)
            pltpu.sync_copy(w_hbm.at[pl.ds(base, GH)], wb)

            def _group(gg, carry2):
                pltpu.async_copy(x_hbm.at[sb.at[gg]], rows, sem).wait()

                def scale16(eb, carry3):
                    wv16 = wb[gg, pl.ds(eb * L, L)]
                    for lane in range(L):
                        wvb = jnp.full((L,), wv16[lane], jnp.float32)
                        e = eb * L + lane
                        for j in range(D // L):
                            rows[e, pl.ds(j * L, L)] = (
                                rows[e, pl.ds(j * L, L)] * wvb)
                    return carry3
                lax.fori_loop(0, GROUP // L, scale16, 0)

                pltpu.sync_copy(rows, acc.at[db.at[gg]], add=True)
                return carry2
            lax.fori_loop(0, GH, _group, 0)
            return carry
        lax.fori_loop(0, NH, _half, 0)

        plsc.subcore_barrier()

        def _pub(k, carry):
            r0 = (pc0 + k) * PCHUNK
            pltpu.sync_copy(acc.at[pl.ds(r0, PCHUNK)], stage)
            pltpu.sync_copy(stage, out_hbm.at[c, pl.ds(r0, PCHUNK)])
            return carry
        lax.fori_loop(0, pcnt, _pub, 0)

    return sc_kernel(x, src, dst, wgt)


def _mm_fused(p, b, W):
    def body(p_ref, b_ref, w_ref, o_ref):
        hs = p_ref[0] + p_ref[1]
        o_ref[...] = jnp.dot(hs, w_ref[...],
                             preferred_element_type=jnp.float32) + b_ref[...]
    return pl.pallas_call(
        body,
        grid=(N // MM_BLK,),
        in_specs=[pl.BlockSpec((NC, MM_BLK, D), lambda i: (0, i, 0)),
                  pl.BlockSpec((1, D), lambda i: (0, 0)),
                  pl.BlockSpec((D, D), lambda i: (0, 0))],
        out_specs=pl.BlockSpec((MM_BLK, D), lambda i: (i, 0)),
        out_shape=jax.ShapeDtypeStruct((N, D), jnp.float32),
    )(p, b, W)


def kernel(x, edge_index, edge_weight, W1, b1, W2, b2):
    pad = E_PAD - E
    spread = (jnp.arange(pad, dtype=jnp.int32) * 37) % N
    src = jnp.concatenate(
        [edge_index[0], spread]).reshape(G_PAD, GROUP)
    dst = jnp.concatenate(
        [edge_index[1], spread]).reshape(G_PAD, GROUP)
    wgt = jnp.concatenate(
        [edge_weight, jnp.zeros((pad,), jnp.float32)]).reshape(G_PAD, GROUP)
    b1r = b1.reshape(1, D)
    b2r = b2.reshape(1, D)

    p1 = _sc_layer(x, src, dst, wgt)
    h1 = _mm_fused(p1, b1r, W1)
    p2 = _sc_layer(h1, src, dst, wgt)
    return _mm_fused(p2, b2r, W2)


# speedup vs baseline: 3.4847x; 1.5611x over previous
import functools

import jax
import jax.numpy as jnp
from jax import lax
from jax.experimental import pallas as pl
from jax.experimental.pallas import tpu as pltpu
from jax.experimental.pallas import tpu_sc as plsc

N = 10000
E = 320000
D = 128
L = 16
GROUP = 128
NC = 2
NS = 16
NW = NC * NS
NGT = 80
NH = 2
GH = NGT // NH
G_PAD = NW * NGT
E_PAD = G_PAD * GROUP
NB = 2
PCHUNK = 80
NPC = N // PCHUNK
PC_TILE = NPC // NS
PC_REM = NPC - PC_TILE * NS
MM_BLK = 2000


def _sc_layer(x, src, dst, wgt):
    mesh = plsc.VectorSubcoreMesh(core_axis_name="c", subcore_axis_name="s")

    @functools.partial(
        pl.kernel,
        out_type=jax.ShapeDtypeStruct((NC, N, D), jnp.float32),
        mesh=mesh,
        scratch_types=[
            pltpu.VMEM_SHARED((N, D), jnp.float32),
            pltpu.VMEM((GH, GROUP), jnp.int32),
            pltpu.VMEM((GH, GROUP), jnp.int32),
            pltpu.VMEM((GH, GROUP), jnp.float32),
            pltpu.VMEM((NB * GROUP, D), jnp.float32),
            pltpu.VMEM((GROUP,), jnp.int32),
            pltpu.VMEM((GROUP,), jnp.int32),
            pltpu.VMEM((GROUP,), jnp.int32),
            pltpu.SemaphoreType.DMA((NB,)),
        ],
    )
    def sc_kernel(x_hbm, src_hbm, dst_hbm, w_hbm, out_hbm,
                  acc, sb, db, wb, rows, sv0, sv1, dv, gsem):
        c = lax.axis_index("c")
        s = lax.axis_index("s")
        wid = c * NS + s
        g0 = wid * NGT

        def _zero(r, carry):
            for j in range(D // L):
                rows[r, pl.ds(j * L, L)] = jnp.zeros((L,), jnp.float32)
            return carry
        lax.fori_loop(0, PCHUNK, _zero, 0)
        pc0 = s * PC_TILE + jnp.minimum(s, PC_REM)
        pcnt = PC_TILE + jnp.where(s < PC_REM, 1, 0)

        def _zacc(k, carry):
            pltpu.sync_copy(rows.at[pl.ds(0, PCHUNK)],
                            acc.at[pl.ds((pc0 + k) * PCHUNK, PCHUNK)])
            return carry
        lax.fori_loop(0, pcnt, _zacc, 0)
        plsc.subcore_barrier()

        def slot(b):
            return rows.at[pl.ds(b * GROUP, GROUP)]

        svs = (sv0, sv1)

        def gather(b):
            pltpu.async_copy(x_hbm.at[svs[b]], slot(b), gsem.at[b])

        def copy_row(blk, gg, dst1d):
            for j in range(D // L):
                dst1d[pl.ds(j * L, L)] = blk[gg, pl.ds(j * L, L)]

        @pl.loop(0, NH)
        def _half(h):
            base = g0 + h * GH
            pltpu.sync_copy(src_hbm.at[pl.ds(base, GH)], sb)
            pltpu.sync_copy(dst_hbm.at[pl.ds(base, GH)], db)
            pltpu.sync_copy(w_hbm.at[pl.ds(base, GH)], wb)

            for b in range(NB):
                copy_row(sb, b, svs[b])
                gather(b)

            @pl.loop(0, GH, step=NB)
            def _groups(k):
                for b in range(NB):
                    gg = k + b
                    pltpu.make_async_copy(
                        x_hbm.at[pl.ds(0, GROUP)], slot(b), gsem.at[b]).wait()

                    def scale16(eb, carry):
                        wv16 = wb[gg, pl.ds(eb * L, L)]
                        for lane in range(L):
                            wv = jnp.full((L,), wv16[lane], jnp.float32)
                            e = b * GROUP + eb * L + lane
                            for j in range(D // L):
                                rows[e, pl.ds(j * L, L)] = (
                                    rows[e, pl.ds(j * L, L)] * wv)
                        return carry
                    lax.fori_loop(0, GROUP // L, scale16, 0)

                    copy_row(db, gg, dv)
                    pltpu.sync_copy(slot(b), acc.at[dv], add=True)

                    @pl.when(gg + NB < GH)
                    def _refill():
                        copy_row(sb, gg + NB, svs[b])
                        gather(b)

        plsc.subcore_barrier()

        def _pub(k, carry):
            r0 = (pc0 + k) * PCHUNK
            pltpu.sync_copy(acc.at[pl.ds(r0, PCHUNK)],
                            rows.at[pl.ds(0, PCHUNK)])
            pltpu.sync_copy(rows.at[pl.ds(0, PCHUNK)],
                            out_hbm.at[c, pl.ds(r0, PCHUNK)])
            return carry
        lax.fori_loop(0, pcnt, _pub, 0)

    return sc_kernel(x, src, dst, wgt)


def _mm_fused(p, b, W):
    def body(p_ref, b_ref, w_ref, o_ref):
        hs = p_ref[0] + p_ref[1]
        o_ref[...] = jnp.dot(hs, w_ref[...],
                             preferred_element_type=jnp.float32) + b_ref[...]
    return pl.pallas_call(
        body,
        grid=(N // MM_BLK,),
        in_specs=[pl.BlockSpec((NC, MM_BLK, D), lambda i: (0, i, 0)),
                  pl.BlockSpec((1, D), lambda i: (0, 0)),
                  pl.BlockSpec((D, D), lambda i: (0, 0))],
        out_specs=pl.BlockSpec((MM_BLK, D), lambda i: (i, 0)),
        out_shape=jax.ShapeDtypeStruct((N, D), jnp.float32),
    )(p, b, W)


def kernel(x, edge_index, edge_weight, W1, b1, W2, b2):
    pad = E_PAD - E
    spread = (jnp.arange(pad, dtype=jnp.int32) * 37) % N
    src = jnp.concatenate(
        [edge_index[0], spread]).reshape(G_PAD, GROUP)
    dst = jnp.concatenate(
        [edge_index[1], spread]).reshape(G_PAD, GROUP)
    wgt = jnp.concatenate(
        [edge_weight, jnp.zeros((pad,), jnp.float32)]).reshape(G_PAD, GROUP)
    b1r = b1.reshape(1, D)
    b2r = b2.reshape(1, D)

    p1 = _sc_layer(x, src, dst, wgt)
    h1 = _mm_fused(p1, b1r, W1)
    p2 = _sc_layer(h1, src, dst, wgt)
    return _mm_fused(p2, b2r, W2)


# speedup vs baseline: 3.5656x; 1.0232x over previous
import functools

import jax
import jax.numpy as jnp
from jax import lax
from jax.experimental import pallas as pl
from jax.experimental.pallas import tpu as pltpu
from jax.experimental.pallas import tpu_sc as plsc

N = 10000
E = 320000
D = 128
L = 16
GROUP = 128
NC = 2
NS = 16
NW = NC * NS
NGT = 80
NH = 2
GH = NGT // NH
G_PAD = NW * NGT
E_PAD = G_PAD * GROUP
NB = 2
PCHUNK = 80
NPC = N // PCHUNK
PC_TILE = NPC // NS
PC_REM = NPC - PC_TILE * NS
MM_BLK = 2000


def _sc_layer(x, src, dst, wgt):
    mesh = plsc.VectorSubcoreMesh(core_axis_name="c", subcore_axis_name="s")

    @functools.partial(
        pl.kernel,
        out_type=jax.ShapeDtypeStruct((NC, N, D), jnp.float32),
        mesh=mesh,
        scratch_types=[
            pltpu.VMEM_SHARED((N, D), jnp.float32),
            pltpu.VMEM((GH, GROUP), jnp.int32),
            pltpu.VMEM((GH, GROUP), jnp.int32),
            pltpu.VMEM((GH, GROUP), jnp.float32),
            pltpu.VMEM((NB * GROUP, D), jnp.float32),
            pltpu.VMEM((64,), jnp.int32),
            pltpu.VMEM((64,), jnp.int32),
            pltpu.SemaphoreType.DMA((NB,)),
            pltpu.SemaphoreType.DMA,
        ],
    )
    def sc_kernel(x_hbm, src_hbm, dst_hbm, w_hbm, out_hbm,
                  acc, sb, db, wb, rows, dva, dvb, gsem, ssem):
        c = lax.axis_index("c")
        s = lax.axis_index("s")
        wid = c * NS + s
        g0 = wid * NGT

        def _zero(r, carry):
            for j in range(D // L):
                rows[r, pl.ds(j * L, L)] = jnp.zeros((L,), jnp.float32)
            return carry
        lax.fori_loop(0, PCHUNK, _zero, 0)
        pc0 = s * PC_TILE + jnp.minimum(s, PC_REM)
        pcnt = PC_TILE + jnp.where(s < PC_REM, 1, 0)

        def _zacc(k, carry):
            pltpu.sync_copy(rows.at[pl.ds(0, PCHUNK)],
                            acc.at[pl.ds((pc0 + k) * PCHUNK, PCHUNK)])
            return carry
        lax.fori_loop(0, pcnt, _zacc, 0)
        plsc.subcore_barrier()

        def slot(b):
            return rows.at[pl.ds(b * GROUP, GROUP)]

        def gather(gg, b):
            pltpu.async_copy(x_hbm.at[sb.at[gg]], slot(b), gsem.at[b])

        @pl.loop(0, NH)
        def _half(h):
            base = g0 + h * GH
            pltpu.sync_copy(src_hbm.at[pl.ds(base, GH)], sb)
            pltpu.sync_copy(dst_hbm.at[pl.ds(base, GH)], db)
            pltpu.sync_copy(w_hbm.at[pl.ds(base, GH)], wb)

            for b in range(NB):
                gather(b, b)

            @pl.loop(0, GH, step=NB)
            def _groups(k):
                for b in range(NB):
                    gg = k + b
                    pltpu.make_async_copy(
                        x_hbm.at[pl.ds(0, GROUP)], slot(b), gsem.at[b]).wait()

                    for j in range(64 // L):
                        dva[pl.ds(j * L, L)] = db[gg, pl.ds(j * L, L)]
                        dvb[pl.ds(j * L, L)] = db[gg, pl.ds(64 + j * L, L)]

                    def scale16(eb, carry):
                        wv16 = wb[gg, pl.ds(eb * L, L)]
                        for lane in range(L):
                            wv = jnp.full((L,), wv16[lane], jnp.float32)
                            e = b * GROUP + eb * L + lane
                            for j in range(D // L):
                                rows[e, pl.ds(j * L, L)] = (
                                    rows[e, pl.ds(j * L, L)] * wv)
                        return carry

                    lax.fori_loop(0, GROUP // (2 * L), scale16, 0)
                    cpa = pltpu.async_copy(
                        rows.at[pl.ds(b * GROUP, 64)], acc.at[dva], ssem,
                        add=True)
                    lax.fori_loop(GROUP // (2 * L), GROUP // L, scale16, 0)
                    pltpu.sync_copy(
                        rows.at[pl.ds(b * GROUP + 64, 64)], acc.at[dvb],
                        add=True)
                    cpa.wait()

                    @pl.when(gg + NB < GH)
                    def _refill():
                        gather(gg + NB, b)

        plsc.subcore_barrier()

        def _pub(k, carry):
            r0 = (pc0 + k) * PCHUNK
            pltpu.sync_copy(acc.at[pl.ds(r0, PCHUNK)],
                            out_hbm.at[c, pl.ds(r0, PCHUNK)])
            return carry
        lax.fori_loop(0, pcnt, _pub, 0)

    return sc_kernel(x, src, dst, wgt)


def _mm_fused(p, b, W):
    def body(p_ref, b_ref, w_ref, o_ref):
        hs = p_ref[0] + p_ref[1]
        o_ref[...] = jnp.dot(hs, w_ref[...],
                             preferred_element_type=jnp.float32) + b_ref[...]
    return pl.pallas_call(
        body,
        grid=(N // MM_BLK,),
        in_specs=[pl.BlockSpec((NC, MM_BLK, D), lambda i: (0, i, 0)),
                  pl.BlockSpec((1, D), lambda i: (0, 0)),
                  pl.BlockSpec((D, D), lambda i: (0, 0))],
        out_specs=pl.BlockSpec((MM_BLK, D), lambda i: (i, 0)),
        out_shape=jax.ShapeDtypeStruct((N, D), jnp.float32),
    )(p, b, W)


def kernel(x, edge_index, edge_weight, W1, b1, W2, b2):
    pad = E_PAD - E
    spread = (jnp.arange(pad, dtype=jnp.int32) * 37) % N
    src = jnp.concatenate(
        [edge_index[0], spread]).reshape(G_PAD, GROUP)
    dst = jnp.concatenate(
        [edge_index[1], spread]).reshape(G_PAD, GROUP)
    wgt = jnp.concatenate(
        [edge_weight, jnp.zeros((pad,), jnp.float32)]).reshape(G_PAD, GROUP)
    b1r = b1.reshape(1, D)
    b2r = b2.reshape(1, D)

    p1 = _sc_layer(x, src, dst, wgt)
    h1 = _mm_fused(p1, b1r, W1)
    p2 = _sc_layer(h1, src, dst, wgt)
    return _mm_fused(p2, b2r, W2)


# speedup vs baseline: 3.7675x; 1.0566x over previous
import functools

import jax
import jax.numpy as jnp
from jax import lax
from jax.experimental import pallas as pl
from jax.experimental.pallas import tpu as pltpu
from jax.experimental.pallas import tpu_sc as plsc

N = 10000
E = 320000
D = 128
L = 16
GROUP = 128
NC = 2
NS = 16
NW = NC * NS
NGT = 80
NH = 2
GH = NGT // NH
G_PAD = NW * NGT
E_PAD = G_PAD * GROUP
NB = 4
SG = 64
PCHUNK = 80
NPC = N // PCHUNK
PC_TILE = NPC // NS
PC_REM = NPC - PC_TILE * NS
MM_BLK = 2000


def _sc_layer(x, src, dst, wgt):
    mesh = plsc.VectorSubcoreMesh(core_axis_name="c", subcore_axis_name="s")

    @functools.partial(
        pl.kernel,
        out_type=jax.ShapeDtypeStruct((NC, N, D), jnp.float32),
        mesh=mesh,
        scratch_types=[
            pltpu.VMEM_SHARED((N, D), jnp.float32),
            pltpu.VMEM((GH, GROUP), jnp.int32),
            pltpu.VMEM((GH, GROUP), jnp.int32),
            pltpu.VMEM((GH, GROUP), jnp.float32),
            pltpu.VMEM((NB * SG, D), jnp.float32),
            pltpu.VMEM((SG,), jnp.int32),
            pltpu.VMEM((SG,), jnp.int32),
            pltpu.VMEM((SG,), jnp.int32),
            pltpu.VMEM((SG,), jnp.int32),
            pltpu.VMEM((SG,), jnp.int32),
            pltpu.VMEM((SG,), jnp.int32),
            pltpu.VMEM((SG,), jnp.int32),
            pltpu.VMEM((SG,), jnp.int32),
            pltpu.SemaphoreType.DMA((NB,)),
            pltpu.SemaphoreType.DMA((NB,)),
        ],
    )
    def sc_kernel(x_hbm, src_hbm, dst_hbm, w_hbm, out_hbm,
                  acc, sb, db, wb, rows, dv0, dv1, dv2, dv3,
                  sv0, sv1, sv2, sv3, gsem, ssem):
        c = lax.axis_index("c")
        s = lax.axis_index("s")
        wid = c * NS + s
        g0 = wid * NGT

        def _zero(r, carry):
            for j in range(D // L):
                rows[r, pl.ds(j * L, L)] = jnp.zeros((L,), jnp.float32)
            return carry
        lax.fori_loop(0, PCHUNK, _zero, 0)
        pc0 = s * PC_TILE + jnp.minimum(s, PC_REM)
        pcnt = PC_TILE + jnp.where(s < PC_REM, 1, 0)

        def _zacc(k, carry):
            pltpu.sync_copy(rows.at[pl.ds(0, PCHUNK)],
                            acc.at[pl.ds((pc0 + k) * PCHUNK, PCHUNK)])
            return carry
        lax.fori_loop(0, pcnt, _zacc, 0)
        plsc.subcore_barrier()

        dvs = (dv0, dv1, dv2, dv3)
        svs = (sv0, sv1, sv2, sv3)
        NSG = 2 * GH

        def slot(b):
            return rows.at[pl.ds(b * SG, SG)]

        def gather(gg, hf, b):
            for j in range(SG // L):
                svs[b][pl.ds(j * L, L)] = sb[gg, pl.ds(hf * SG + j * L, L)]
            pltpu.async_copy(x_hbm.at[svs[b]], slot(b), gsem.at[b])

        @pl.loop(0, NH)
        def _half(h):
            base = g0 + h * GH
            pltpu.sync_copy(src_hbm.at[pl.ds(base, GH)], sb)
            pltpu.sync_copy(dst_hbm.at[pl.ds(base, GH)], db)
            pltpu.sync_copy(w_hbm.at[pl.ds(base, GH)], wb)

            for b in range(NB):
                gather(b // 2, b % 2, b)

            @pl.loop(0, NSG, step=NB)
            def _groups(k):
                def process(b):
                    hf = b % 2
                    gg = k // 2 + b // 2
                    pltpu.make_async_copy(
                        x_hbm.at[pl.ds(0, SG)], slot(b), gsem.at[b]).wait()

                    for j in range(SG // L):
                        dvs[b][pl.ds(j * L, L)] = (
                            db[gg, pl.ds(hf * SG + j * L, L)])

                    def scale16(eb, carry):
                        wv16 = wb[gg, pl.ds(hf * SG + eb * L, L)]
                        for lane in range(L):
                            wv = jnp.full((L,), wv16[lane], jnp.float32)
                            e = b * SG + eb * L + lane
                            for j in range(D // L):
                                rows[e, pl.ds(j * L, L)] = (
                                    rows[e, pl.ds(j * L, L)] * wv)
                        return carry
                    lax.fori_loop(0, SG // L, scale16, 0)

                    return pltpu.async_copy(
                        slot(b), acc.at[dvs[b]], ssem.at[b], add=True)

                def drain_refill(b, desc):
                    desc.wait()

                    @pl.when(k + NB + b < NSG)
                    def _refill():
                        gather(k // 2 + (NB + b) // 2, b % 2, b)

                d0 = process(0)
                d1 = process(1)
                drain_refill(0, d0)
                drain_refill(1, d1)
                d2 = process(2)
                d3 = process(3)
                drain_refill(2, d2)
                drain_refill(3, d3)

        plsc.subcore_barrier()

        def _pub(k, carry):
            r0 = (pc0 + k) * PCHUNK
            pltpu.sync_copy(acc.at[pl.ds(r0, PCHUNK)],
                            out_hbm.at[c, pl.ds(r0, PCHUNK)])
            return carry
        lax.fori_loop(0, pcnt, _pub, 0)

    return sc_kernel(x, src, dst, wgt)


def _mm_fused(p, b, W):
    def body(p_ref, b_ref, w_ref, o_ref):
        hs = p_ref[0] + p_ref[1]
        o_ref[...] = jnp.dot(hs, w_ref[...],
                             preferred_element_type=jnp.float32) + b_ref[...]
    return pl.pallas_call(
        body,
        grid=(N // MM_BLK,),
        in_specs=[pl.BlockSpec((NC, MM_BLK, D), lambda i: (0, i, 0)),
                  pl.BlockSpec((1, D), lambda i: (0, 0)),
                  pl.BlockSpec((D, D), lambda i: (0, 0))],
        out_specs=pl.BlockSpec((MM_BLK, D), lambda i: (i, 0)),
        out_shape=jax.ShapeDtypeStruct((N, D), jnp.float32),
    )(p, b, W)


def kernel(x, edge_index, edge_weight, W1, b1, W2, b2):
    pad = E_PAD - E
    spread = (jnp.arange(pad, dtype=jnp.int32) * 37) % N
    src = jnp.concatenate(
        [edge_index[0], spread]).reshape(G_PAD, GROUP)
    dst = jnp.concatenate(
        [edge_index[1], spread]).reshape(G_PAD, GROUP)
    wgt = jnp.concatenate(
        [edge_weight, jnp.zeros((pad,), jnp.float32)]).reshape(G_PAD, GROUP)
    b1r = b1.reshape(1, D)
    b2r = b2.reshape(1, D)

    p1 = _sc_layer(x, src, dst, wgt)
    h1 = _mm_fused(p1, b1r, W1)
    p2 = _sc_layer(h1, src, dst, wgt)
    return _mm_fused(p2, b2r, W2)
